# baseline (device time: 15237 ns/iter reference)
import jax
import jax.numpy as jnp
from jax import lax
from jax.experimental import pallas as pl
from jax.experimental.pallas import tpu as pltpu

N_DEV = 4


def kernel(x, W, labels):
    T, D = x.shape
    _, V = W.shape
    labels2d = labels.reshape(T, 1)

    COMM = True

    def body(x_ref, w_ref, lab_ref, out_ref, comm_ref, send_sems, recv_sems):
        my_pos = lax.axis_index("i")

        if COMM:
            barrier_sem = pltpu.get_barrier_semaphore()
            for j in range(1, N_DEV):
                peer = lax.rem(my_pos + j, N_DEV)
                pl.semaphore_signal(
                    barrier_sem, inc=1,
                    device_id=(peer,), device_id_type=pl.DeviceIdType.MESH,
                )

        xb = x_ref[...].astype(jnp.bfloat16)
        wb = w_ref[...].astype(jnp.bfloat16)
        logits = jnp.dot(
            xb, wb, preferred_element_type=jnp.float32
        ).astype(jnp.bfloat16)

        e = jnp.exp(logits)
        local_lab = lab_ref[...] - my_pos * V
        iota = lax.broadcasted_iota(jnp.int32, (T, V), 1)
        onehot = iota == local_lab
        masked = jnp.where(onehot, logits, jnp.bfloat16(0.0))

        ones = jnp.ones((V, 128), jnp.bfloat16)
        s128 = jnp.dot(e, ones, preferred_element_type=jnp.float32)
        c128 = jnp.dot(masked, ones, preferred_element_type=jnp.float32)
        comm_ref[0] = jnp.concatenate([s128[:, :1], c128[:, :1]], axis=1)

        rdmas = []
        if COMM:
            pl.semaphore_wait(barrier_sem, N_DEV - 1)

            for j in range(1, N_DEV):
                peer = lax.rem(my_pos + j, N_DEV)
                rdma = pltpu.make_async_remote_copy(
                    src_ref=comm_ref.at[0],
                    dst_ref=comm_ref.at[j],
                    send_sem=send_sems.at[j - 1],
                    recv_sem=recv_sems.at[j - 1],
                    device_id=(peer,),
                    device_id_type=pl.DeviceIdType.MESH,
                )
                rdma.start()
                rdmas.append(rdma)
            for rdma in rdmas:
                rdma.wait_recv()

        totals = jnp.sum(comm_ref[...], axis=0)
        out_ref[...] = jnp.log(totals[:, 0:1]) - totals[:, 1:2]

        for rdma in rdmas:
            rdma.wait_send()

    out = pl.pallas_call(
        body,
        out_shape=jax.ShapeDtypeStruct((T, 1), jnp.float32),
        in_specs=[
            pl.BlockSpec(memory_space=pltpu.VMEM),
            pl.BlockSpec(memory_space=pltpu.VMEM),
            pl.BlockSpec(memory_space=pltpu.VMEM),
        ],
        out_specs=pl.BlockSpec(memory_space=pltpu.VMEM),
        scratch_shapes=[
            pltpu.VMEM((N_DEV, T, 2), jnp.float32),
            pltpu.SemaphoreType.DMA((N_DEV - 1,)),
            pltpu.SemaphoreType.DMA((N_DEV - 1,)),
        ],
        compiler_params=(
            pltpu.CompilerParams(collective_id=0) if COMM else None
        ),
    )(x, W, labels2d)
    return out.reshape(T)


# device time: 14859 ns/iter; 1.0254x vs baseline; 1.0254x over previous
import jax
import jax.numpy as jnp
from jax import lax
from jax.experimental import pallas as pl
from jax.experimental.pallas import tpu as pltpu

N_DEV = 4
VC = 1024


def kernel(x, W, labels):
    T, D = x.shape
    _, V = W.shape
    K = V // VC

    def body(x_ref, w_ref, lab_ref, out_ref,
             xt_ref, acc_ref, comm_ref, send_sems, recv_sems):
        k = pl.program_id(0)
        my_pos = lax.axis_index("i")
        barrier_sem = pltpu.get_barrier_semaphore()

        @pl.when(k == 0)
        def _():
            for j in range(1, N_DEV):
                peer = lax.rem(my_pos + j, N_DEV)
                pl.semaphore_signal(
                    barrier_sem, inc=1,
                    device_id=(peer,), device_id_type=pl.DeviceIdType.MESH,
                )
            xt_ref[...] = jnp.transpose(
                x_ref[...].astype(jnp.bfloat16), (1, 0)
            )

        wb = w_ref[...].astype(jnp.bfloat16)
        logits_t = lax.dot_general(
            wb, xt_ref[...],
            dimension_numbers=(((0,), (0,)), ((), ())),
            preferred_element_type=jnp.float32,
        ).astype(jnp.bfloat16)

        e_t = jnp.exp(logits_t)
        vio = lax.broadcasted_iota(jnp.int32, (VC, T), 0) + (
            my_pos * V + k * VC
        )
        onehot = vio == lab_ref[...].reshape(1, T)
        masked_t = jnp.where(onehot, logits_t, jnp.bfloat16(0.0))

        ones8 = jnp.ones((8, VC), jnp.bfloat16)
        s8 = lax.dot_general(
            ones8, e_t, dimension_numbers=(((1,), (0,)), ((), ())),
            preferred_element_type=jnp.float32,
        )
        c8 = lax.dot_general(
            ones8, masked_t, dimension_numbers=(((1,), (0,)), ((), ())),
            preferred_element_type=jnp.float32,
        )
        part = jnp.concatenate([s8[0:1], c8[0:1]], axis=0)

        @pl.when(k == 0)
        def _():
            acc_ref[...] = part

        @pl.when(k > 0)
        def _():
            acc_ref[...] = acc_ref[...] + part

        @pl.when(k == K - 1)
        def _():
            comm_ref[0] = acc_ref[...]
            pl.semaphore_wait(barrier_sem, N_DEV - 1)

            rdmas = []
            for j in range(1, N_DEV):
                peer = lax.rem(my_pos + j, N_DEV)
                rdma = pltpu.make_async_remote_copy(
                    src_ref=comm_ref.at[0],
                    dst_ref=comm_ref.at[j],
                    send_sem=send_sems.at[j - 1],
                    recv_sem=recv_sems.at[j - 1],
                    device_id=(peer,),
                    device_id_type=pl.DeviceIdType.MESH,
                )
                rdma.start()
                rdmas.append(rdma)
            for rdma in rdmas:
                rdma.wait_recv()

            tot = (
                comm_ref[0] + comm_ref[1] + comm_ref[2] + comm_ref[3]
            )
            out_ref[...] = (jnp.log(tot[0:1]) - tot[1:2]).reshape(T)

            for rdma in rdmas:
                rdma.wait_send()

    out = pl.pallas_call(
        body,
        grid=(K,),
        out_shape=jax.ShapeDtypeStruct((T,), jnp.float32),
        in_specs=[
            pl.BlockSpec((T, D), lambda k: (0, 0)),
            pl.BlockSpec((D, VC), lambda k: (0, k)),
            pl.BlockSpec((T,), lambda k: (0,)),
        ],
        out_specs=pl.BlockSpec((T,), lambda k: (0,)),
        scratch_shapes=[
            pltpu.VMEM((D, T), jnp.bfloat16),
            pltpu.VMEM((2, T), jnp.float32),
            pltpu.VMEM((N_DEV, 2, T), jnp.float32),
            pltpu.SemaphoreType.DMA((N_DEV - 1,)),
            pltpu.SemaphoreType.DMA((N_DEV - 1,)),
        ],
        compiler_params=pltpu.CompilerParams(collective_id=0),
    )(x, W, labels)
    return out


# device time: 13095 ns/iter; 1.1636x vs baseline; 1.1347x over previous
import jax
import jax.numpy as jnp
from jax import lax
from jax.experimental import pallas as pl
from jax.experimental.pallas import tpu as pltpu

N_DEV = 4


def kernel(x, W, labels):
    T, D = x.shape
    _, V = W.shape

    def body(x_ref, w_ref, lab_ref, out_ref, comm_ref, send_sems, recv_sems):
        my_pos = lax.axis_index("i")
        barrier_sem = pltpu.get_barrier_semaphore()
        for j in range(1, N_DEV):
            peer = lax.rem(my_pos + j, N_DEV)
            pl.semaphore_signal(
                barrier_sem, inc=1,
                device_id=(peer,), device_id_type=pl.DeviceIdType.MESH,
            )

        xb = x_ref[...].astype(jnp.bfloat16)
        wb = w_ref[...].astype(jnp.bfloat16)
        logits_t = lax.dot_general(
            wb, xb,
            dimension_numbers=(((0,), (1,)), ((), ())),
            preferred_element_type=jnp.float32,
        ).astype(jnp.bfloat16)

        e_t = jnp.exp(logits_t)
        vio = lax.broadcasted_iota(jnp.int32, (V, T), 0) + my_pos * V
        lab_row = lab_ref[...].reshape(1, T)
        masked_t = jnp.where(vio == lab_row, logits_t, jnp.bfloat16(0.0))

        ones8 = jnp.ones((8, V), jnp.bfloat16)
        s8 = lax.dot_general(
            ones8, e_t, dimension_numbers=(((1,), (0,)), ((), ())),
            preferred_element_type=jnp.float32,
        )
        c8 = lax.dot_general(
            ones8, masked_t, dimension_numbers=(((1,), (0,)), ((), ())),
            preferred_element_type=jnp.float32,
        )
        comm_ref[0] = jnp.concatenate([s8[0:1], c8[0:1]], axis=0)

        pl.semaphore_wait(barrier_sem, N_DEV - 1)

        rdmas = []
        for j in range(1, N_DEV):
            peer = lax.rem(my_pos + j, N_DEV)
            rdma = pltpu.make_async_remote_copy(
                src_ref=comm_ref.at[0],
                dst_ref=comm_ref.at[j],
                send_sem=send_sems.at[j - 1],
                recv_sem=recv_sems.at[j - 1],
                device_id=(peer,),
                device_id_type=pl.DeviceIdType.MESH,
            )
            rdma.start()
            rdmas.append(rdma)
        for rdma in rdmas:
            rdma.wait_recv()

        tot = comm_ref[0] + comm_ref[1] + comm_ref[2] + comm_ref[3]
        out_ref[...] = (jnp.log(tot[0:1]) - tot[1:2]).reshape(T)

        for rdma in rdmas:
            rdma.wait_send()

    out = pl.pallas_call(
        body,
        out_shape=jax.ShapeDtypeStruct((T,), jnp.float32),
        in_specs=[
            pl.BlockSpec(memory_space=pltpu.MemorySpace.VMEM),
            pl.BlockSpec(memory_space=pltpu.MemorySpace.VMEM),
            pl.BlockSpec(memory_space=pltpu.MemorySpace.VMEM),
        ],
        out_specs=pl.BlockSpec(memory_space=pltpu.MemorySpace.VMEM),
        scratch_shapes=[
            pltpu.VMEM((N_DEV, 2, T), jnp.float32),
            pltpu.SemaphoreType.DMA((N_DEV - 1,)),
            pltpu.SemaphoreType.DMA((N_DEV - 1,)),
        ],
        compiler_params=pltpu.CompilerParams(collective_id=0),
    )(x, W, labels)
    return out
